# baseline (speedup 1.0000x reference)
import functools

import jax
import jax.numpy as jnp
from jax import lax
from jax.experimental import pallas as pl
from jax.experimental.pallas import tpu as pltpu
from jax.experimental.pallas import tpu_sc as plsc

_ROWS = 4 * 2048
_COLS = 4096
_BLOCK_ROWS = 256

_SC_ROWS = 1024
_NW = 32
_SC_CHUNK = 16384
_SC_UNROLL = 4

_SC_N = _SC_ROWS * _COLS
_SC_PER_W = _SC_N // _NW
_SC_NCHUNK = _SC_PER_W // _SC_CHUNK

_TC_GRID = (_ROWS - _SC_ROWS) // _BLOCK_ROWS
_TC_OFF = _SC_ROWS // _BLOCK_ROWS


def _tc_kernel(p_ref, t_ref, m_ref, out_ref, acc_ref):
    i = pl.program_id(0)

    @pl.when(i == 0)
    def _init():
        acc_ref[0] = 0.0
        acc_ref[1] = 0.0

    mf = (m_ref[...] == 1).astype(jnp.float32)
    d = (p_ref[...] - t_ref[...]) * mf
    acc_ref[0] += jnp.sum(d * d)
    acc_ref[1] += jnp.sum(mf)

    @pl.when(i == _TC_GRID - 1)
    def _fini():
        out_ref[0] = acc_ref[0]
        out_ref[1] = acc_ref[1]


def _tc_part(p2, t2, m2):
    in_spec = pl.BlockSpec((_BLOCK_ROWS, _COLS), lambda i: (i + _TC_OFF, 0))
    return pl.pallas_call(
        _tc_kernel,
        grid=(_TC_GRID,),
        in_specs=[in_spec, in_spec, in_spec],
        out_specs=pl.BlockSpec(memory_space=pltpu.SMEM),
        out_shape=jax.ShapeDtypeStruct((2,), jnp.float32),
        scratch_shapes=[pltpu.SMEM((2,), jnp.float32)],
    )(p2, t2, m2)


def _make_sc_part():
    mesh = plsc.VectorSubcoreMesh(core_axis_name="c", subcore_axis_name="s")

    @functools.partial(
        pl.kernel,
        mesh=mesh,
        out_type=[
            jax.ShapeDtypeStruct((_NW, 16), jnp.float32),
            jax.ShapeDtypeStruct((_NW, 16), jnp.float32),
        ],
        scratch_types=[
            pltpu.VMEM((_SC_CHUNK,), jnp.float32),
            pltpu.VMEM((_SC_CHUNK,), jnp.float32),
            pltpu.VMEM((_SC_CHUNK,), jnp.int32),
            pltpu.VMEM((_SC_CHUNK,), jnp.float32),
            pltpu.VMEM((_SC_CHUNK,), jnp.float32),
            pltpu.VMEM((_SC_CHUNK,), jnp.int32),
            pltpu.VMEM((16,), jnp.float32),
            pltpu.VMEM((16,), jnp.float32),
            pltpu.SemaphoreType.DMA,
            pltpu.SemaphoreType.DMA,
        ],
    )
    def sc_kernel(p_hbm, t_hbm, m_hbm, sq_out, cnt_out,
                  pb0, tb0, mb0, pb1, tb1, mb1, sqv, cnv, sem0, sem1):
        wid = lax.axis_index("s") * 2 + lax.axis_index("c")
        base = wid * _SC_PER_W
        bufs = ((pb0, tb0, mb0, sem0), (pb1, tb1, mb1, sem1))

        def issue(c, b):
            off = pl.multiple_of(base + c * _SC_CHUNK, _SC_CHUNK)
            pb, tb, mb, sem = bufs[b]
            pltpu.async_copy(p_hbm.at[pl.ds(off, _SC_CHUNK)], pb, sem)
            pltpu.async_copy(t_hbm.at[pl.ds(off, _SC_CHUNK)], tb, sem)
            pltpu.async_copy(m_hbm.at[pl.ds(off, _SC_CHUNK)], mb, sem)

        def wait(b):
            pb, tb, mb, sem = bufs[b]
            pltpu.make_async_copy(p_hbm.at[pl.ds(0, _SC_CHUNK)], pb, sem).wait()
            pltpu.make_async_copy(t_hbm.at[pl.ds(0, _SC_CHUNK)], tb, sem).wait()
            pltpu.make_async_copy(m_hbm.at[pl.ds(0, _SC_CHUNK)], mb, sem).wait()

        def accum(b, carry):
            pb, tb, mb, _ = bufs[b]

            def inner(i, acc):
                accs = list(acc)
                off0 = i * (16 * _SC_UNROLL)
                for u in range(_SC_UNROLL):
                    sl = pl.ds(off0 + 16 * u, 16)
                    d = pb[sl] - tb[sl]
                    sel = mb[sl] == 1
                    df = jnp.where(sel, d, jnp.float32(0))
                    accs[u] = accs[u] + df * df
                    accs[_SC_UNROLL + u] = accs[_SC_UNROLL + u] + jnp.where(
                        sel, jnp.float32(1), jnp.float32(0))
                return tuple(accs)

            return lax.fori_loop(0, _SC_CHUNK // (16 * _SC_UNROLL), inner, carry)

        issue(0, 0)
        issue(1, 1)
        zero = jnp.zeros((16,), jnp.float32)
        carry = (zero,) * (2 * _SC_UNROLL)

        def pair(cp, carry):
            c0 = cp * 2
            wait(0)
            carry = accum(0, carry)

            @pl.when(c0 + 2 < _SC_NCHUNK)
            def _():
                issue(c0 + 2, 0)

            wait(1)
            carry = accum(1, carry)

            @pl.when(c0 + 3 < _SC_NCHUNK)
            def _():
                issue(c0 + 3, 1)

            return carry

        carry = lax.fori_loop(0, _SC_NCHUNK // 2, pair, carry)

        sqv[...] = carry[0] + carry[1] + carry[2] + carry[3]
        cnv[...] = carry[4] + carry[5] + carry[6] + carry[7]
        pltpu.sync_copy(sqv, sq_out.at[wid])
        pltpu.sync_copy(cnv, cnt_out.at[wid])

    return sc_kernel


_sc_part = _make_sc_part()


def kernel(pred, target, mask):
    m32 = mask.astype(jnp.int32)
    p2 = pred.reshape(_ROWS, _COLS)
    t2 = target.reshape(_ROWS, _COLS)
    m2 = m32.reshape(_ROWS, _COLS)

    pf = pred.reshape(-1)
    tf = target.reshape(-1)
    mf = m32.reshape(-1)

    sc_sq, sc_cnt = _sc_part(pf, tf, mf)
    tc_out = _tc_part(p2, t2, m2)

    sq_sum = tc_out[0] + jnp.sum(sc_sq)
    count = tc_out[1] + jnp.sum(sc_cnt)
    return sq_sum / count


# speedup vs baseline: 3.0001x; 3.0001x over previous
import functools

import jax
import jax.numpy as jnp
from jax import lax
from jax.experimental import pallas as pl
from jax.experimental.pallas import tpu as pltpu
from jax.experimental.pallas import tpu_sc as plsc

_ROWS = 4 * 2048
_COLS = 4096
_BLOCK_ROWS = 256

_SC_ROWS = 1024
_NW = 32
_SC_UNROLL = 4

_SLAB_R = 8
_SLAB_C = 2048
_ROWS_PER_W = _SC_ROWS // _NW
_CHUNKS_PER_W = (_ROWS_PER_W // _SLAB_R) * 2

_TC_GRID = (_ROWS - _SC_ROWS) // _BLOCK_ROWS
_TC_OFF = _SC_ROWS // _BLOCK_ROWS


def _tc_kernel(p_ref, t_ref, m_ref, out_ref, acc_ref):
    i = pl.program_id(0)

    @pl.when(i == 0)
    def _init():
        acc_ref[0] = 0.0
        acc_ref[1] = 0.0

    mf = (m_ref[...] == 1).astype(jnp.float32)
    d = (p_ref[...] - t_ref[...]) * mf
    acc_ref[0] += jnp.sum(d * d)
    acc_ref[1] += jnp.sum(mf)

    @pl.when(i == _TC_GRID - 1)
    def _fini():
        out_ref[0] = acc_ref[0]
        out_ref[1] = acc_ref[1]


def _tc_part(p2, t2, m2):
    in_spec = pl.BlockSpec((_BLOCK_ROWS, _COLS), lambda i: (i + _TC_OFF, 0))
    return pl.pallas_call(
        _tc_kernel,
        grid=(_TC_GRID,),
        in_specs=[in_spec, in_spec, in_spec],
        out_specs=pl.BlockSpec(memory_space=pltpu.SMEM),
        out_shape=jax.ShapeDtypeStruct((2,), jnp.float32),
        scratch_shapes=[pltpu.SMEM((2,), jnp.float32)],
    )(p2, t2, m2)


def _make_sc_part():
    mesh = plsc.VectorSubcoreMesh(core_axis_name="c", subcore_axis_name="s")

    @functools.partial(
        pl.kernel,
        mesh=mesh,
        out_type=[
            jax.ShapeDtypeStruct((_NW, 16), jnp.float32),
            jax.ShapeDtypeStruct((_NW, 16), jnp.float32),
        ],
        scratch_types=[
            pltpu.VMEM((_SLAB_R, _SLAB_C), jnp.float32),
            pltpu.VMEM((_SLAB_R, _SLAB_C), jnp.float32),
            pltpu.VMEM((_SLAB_R, _SLAB_C), jnp.int32),
            pltpu.VMEM((_SLAB_R, _SLAB_C), jnp.float32),
            pltpu.VMEM((_SLAB_R, _SLAB_C), jnp.float32),
            pltpu.VMEM((_SLAB_R, _SLAB_C), jnp.int32),
            pltpu.VMEM((16,), jnp.float32),
            pltpu.VMEM((16,), jnp.float32),
            pltpu.SemaphoreType.DMA,
            pltpu.SemaphoreType.DMA,
        ],
    )
    def sc_kernel(p_hbm, t_hbm, m_hbm, sq_out, cnt_out,
                  pb0, tb0, mb0, pb1, tb1, mb1, sqv, cnv, sem0, sem1):
        wid = lax.axis_index("s") * 2 + lax.axis_index("c")
        row0 = wid * _ROWS_PER_W
        bufs = ((pb0, tb0, mb0, sem0), (pb1, tb1, mb1, sem1))

        def issue(k):
            r0 = pl.multiple_of(row0 + (k // 2) * _SLAB_R, _SLAB_R)
            c0 = (k % 2) * _SLAB_C
            pb, tb, mb, sem = bufs[k % 2]
            sl = (pl.ds(r0, _SLAB_R), pl.ds(c0, _SLAB_C))
            pltpu.async_copy(p_hbm.at[sl], pb, sem)
            pltpu.async_copy(t_hbm.at[sl], tb, sem)
            pltpu.async_copy(m_hbm.at[sl], mb, sem)

        def wait(b):
            pb, tb, mb, sem = bufs[b]
            sl = (pl.ds(0, _SLAB_R), pl.ds(0, _SLAB_C))
            pltpu.make_async_copy(p_hbm.at[sl], pb, sem).wait()
            pltpu.make_async_copy(t_hbm.at[sl], tb, sem).wait()
            pltpu.make_async_copy(m_hbm.at[sl], mb, sem).wait()

        def accum(b, carry):
            pb, tb, mb, _ = bufs[b]

            def row_body(r, acc):
                prow = pb.at[r]
                trow = tb.at[r]
                mrow = mb.at[r]

                def inner(i, acc2):
                    accs = list(acc2)
                    off0 = i * (16 * _SC_UNROLL)
                    for u in range(_SC_UNROLL):
                        sl = pl.ds(off0 + 16 * u, 16)
                        d = prow[sl] - trow[sl]
                        sel = mrow[sl] == 1
                        df = jnp.where(sel, d, jnp.float32(0))
                        accs[u] = accs[u] + df * df
                        accs[_SC_UNROLL + u] = accs[_SC_UNROLL + u] + jnp.where(
                            sel, jnp.float32(1), jnp.float32(0))
                    return tuple(accs)

                return lax.fori_loop(
                    0, _SLAB_C // (16 * _SC_UNROLL), inner, acc)

            return lax.fori_loop(0, _SLAB_R, row_body, carry)

        issue(0)
        issue(1)
        zero = jnp.zeros((16,), jnp.float32)
        carry = (zero,) * (2 * _SC_UNROLL)

        for k in range(_CHUNKS_PER_W):
            wait(k % 2)
            carry = accum(k % 2, carry)
            if k + 2 < _CHUNKS_PER_W:
                issue(k + 2)

        sqv[...] = carry[0] + carry[1] + carry[2] + carry[3]
        cnv[...] = carry[4] + carry[5] + carry[6] + carry[7]
        pltpu.sync_copy(sqv, sq_out.at[wid])
        pltpu.sync_copy(cnv, cnt_out.at[wid])

    return sc_kernel


_sc_part = _make_sc_part()


def kernel(pred, target, mask):
    m32 = mask.astype(jnp.int32)
    p2 = pred.reshape(_ROWS, _COLS)
    t2 = target.reshape(_ROWS, _COLS)
    m2 = m32.reshape(_ROWS, _COLS)

    sc_sq, sc_cnt = _sc_part(p2, t2, m2)
    tc_out = _tc_part(p2, t2, m2)

    sq_sum = tc_out[0] + jnp.sum(sc_sq)
    count = tc_out[1] + jnp.sum(sc_cnt)
    return sq_sum / count


# speedup vs baseline: 3.6014x; 1.2004x over previous
import jax
import jax.numpy as jnp
from jax.experimental import pallas as pl
from jax.experimental.pallas import tpu as pltpu

_ROWS = 4 * 2048
_COLS = 4096
_BLOCK_ROWS = 256
_GRID = _ROWS // _BLOCK_ROWS


def _mse_kernel(p_ref, t_ref, m_ref, out_ref, acc_ref):
    i = pl.program_id(0)

    @pl.when(i == 0)
    def _init():
        acc_ref[0] = 0.0
        acc_ref[1] = 0.0

    mf = (m_ref[...] == 1).astype(jnp.float32)
    d = (p_ref[...] - t_ref[...]) * mf
    acc_ref[0] += jnp.sum(d * d)
    acc_ref[1] += jnp.sum(mf)

    @pl.when(i == _GRID - 1)
    def _fini():
        out_ref[0] = acc_ref[0] / acc_ref[1]


def kernel(pred, target, mask):
    p2 = pred.reshape(_ROWS, _COLS)
    t2 = target.reshape(_ROWS, _COLS)
    m2 = mask.astype(jnp.int32).reshape(_ROWS, _COLS)

    in_spec = pl.BlockSpec((_BLOCK_ROWS, _COLS), lambda i: (i, 0))
    out = pl.pallas_call(
        _mse_kernel,
        grid=(_GRID,),
        in_specs=[in_spec, in_spec, in_spec],
        out_specs=pl.BlockSpec(memory_space=pltpu.SMEM),
        out_shape=jax.ShapeDtypeStruct((1,), jnp.float32),
        scratch_shapes=[pltpu.SMEM((2,), jnp.float32)],
    )(p2, t2, m2)
    return out[0]


# speedup vs baseline: 3.6059x; 1.0013x over previous
import jax
import jax.numpy as jnp
from jax.experimental import pallas as pl
from jax.experimental.pallas import tpu as pltpu

_ROWS = 4 * 2048
_COLS = 4096
_BLOCK_ROWS = 128
_GRID = _ROWS // (2 * _BLOCK_ROWS)


def _mse_kernel(p0, p1, t0, t1, m0, m1, out_ref, acc_ref):
    i = pl.program_id(0)

    @pl.when(i == 0)
    def _init():
        acc_ref[0] = 0.0
        acc_ref[1] = 0.0

    s = 0.0
    c = 0.0
    for p_ref, t_ref, m_ref in ((p0, t0, m0), (p1, t1, m1)):
        mf = (m_ref[...] == 1).astype(jnp.float32)
        d = (p_ref[...] - t_ref[...]) * mf
        s += jnp.sum(d * d)
        c += jnp.sum(mf)
    acc_ref[0] += s
    acc_ref[1] += c

    @pl.when(i == _GRID - 1)
    def _fini():
        out_ref[0] = acc_ref[0] / acc_ref[1]


def kernel(pred, target, mask):
    p2 = pred.reshape(_ROWS, _COLS)
    t2 = target.reshape(_ROWS, _COLS)
    m2 = mask.astype(jnp.int32).reshape(_ROWS, _COLS)

    lo = pl.BlockSpec((_BLOCK_ROWS, _COLS), lambda i: (2 * i, 0))
    hi = pl.BlockSpec((_BLOCK_ROWS, _COLS), lambda i: (2 * i + 1, 0))
    out = pl.pallas_call(
        _mse_kernel,
        grid=(_GRID,),
        in_specs=[lo, hi, lo, hi, lo, hi],
        out_specs=pl.BlockSpec(memory_space=pltpu.SMEM),
        out_shape=jax.ShapeDtypeStruct((1,), jnp.float32),
        scratch_shapes=[pltpu.SMEM((2,), jnp.float32)],
    )(p2, p2, t2, t2, m2, m2)
    return out[0]
